# baseline (speedup 1.0000x reference)
import functools

import jax
import jax.numpy as jnp
from jax import lax
from jax.experimental import pallas as pl
from jax.experimental.pallas import tpu as pltpu, tpu_sc as plsc

N = 10000
E = 320000
D = 128

NC = 2
NS = 16
NW = NC * NS
L = 16

EPT = E // NW
C = 80
EPTP = 10240
NCHUNK = EPTP // C
NPAIR = NCHUNK // 2 - 1
ZROWS = 8
NPAD = 10240
RPT = NPAD // NS
TRASH = N

_mesh = plsc.VectorSubcoreMesh(
    core_axis_name="c", subcore_axis_name="s", num_cores=NC, num_subcores=NS
)


@functools.partial(
    pl.kernel,
    out_type=jax.ShapeDtypeStruct((NC, NPAD, L), jnp.float32),
    mesh=_mesh,
    scratch_types=[
        pltpu.VMEM((NCHUNK, C), jnp.int32),
        pltpu.VMEM((C, L), jnp.float32),
        pltpu.VMEM((ZROWS, L), jnp.float32),
        pltpu.VMEM_SHARED((NPAD, L), jnp.float32),
    ],
)
def _deg_kernel(col_hbm, out_hbm, cidx_v, ones_v, z_v, acc):
    c = lax.axis_index("c")
    s = lax.axis_index("s")
    wid = s * NC + c
    for i in range(C):
        ones_v[i, :] = jnp.ones((L,), jnp.float32)
    for i in range(ZROWS):
        z_v[i, :] = jnp.zeros((L,), jnp.float32)
    for r in range(RPT // ZROWS):
        pltpu.sync_copy(z_v, acc.at[pl.ds(s * RPT + r * ZROWS, ZROWS)])
    pltpu.sync_copy(col_hbm.at[wid], cidx_v)
    plsc.subcore_barrier()

    def body(j, carry):
        pltpu.sync_copy(ones_v, acc.at[cidx_v.at[j]], add=True)
        return carry

    lax.fori_loop(0, NCHUNK, body, 0)
    plsc.subcore_barrier()
    pltpu.sync_copy(acc.at[pl.ds(s * RPT, RPT)],
                    out_hbm.at[c, pl.ds(s * RPT, RPT)])


@functools.partial(
    pl.kernel,
    out_type=jax.ShapeDtypeStruct((NC, NPAD, D), jnp.float32),
    mesh=_mesh,
    scratch_types=[
        pltpu.VMEM((NCHUNK, C), jnp.int32),
        pltpu.VMEM((NCHUNK, C), jnp.int32),
        pltpu.VMEM((C, D), jnp.float32),
        pltpu.VMEM((ZROWS, D), jnp.float32),
        pltpu.VMEM_SHARED((NPAD, D), jnp.float32),
        pltpu.SemaphoreType.DMA,
    ],
)
def _agg_kernel(hs_hbm, row_hbm, col_hbm, out_hbm,
                ridx_v, cidx_v, rows0, z_v, acc, sem0):
    c = lax.axis_index("c")
    s = lax.axis_index("s")
    wid = s * NC + c
    for i in range(ZROWS):
        for j in range(D // L):
            z_v[i, j * L:(j + 1) * L] = jnp.zeros((L,), jnp.float32)
    for r in range(RPT // ZROWS):
        pltpu.sync_copy(z_v, acc.at[pl.ds(s * RPT + r * ZROWS, ZROWS)])
    pltpu.sync_copy(row_hbm.at[wid], ridx_v)
    pltpu.sync_copy(col_hbm.at[wid], cidx_v)
    plsc.subcore_barrier()

    def body(j, carry):
        pltpu.async_copy(hs_hbm.at[ridx_v.at[j]], rows0, sem0).wait()
        pltpu.sync_copy(rows0, acc.at[cidx_v.at[j]], add=True)
        return carry

    lax.fori_loop(0, NCHUNK, body, 0)
    plsc.subcore_barrier()
    pltpu.sync_copy(acc.at[pl.ds(s * RPT, RPT)],
                    out_hbm.at[c, pl.ds(s * RPT, RPT)])


def _dinv_from_degp(degp):
    deg = degp[0, :N, 0:1] + degp[1, :N, 0:1] + 1.0
    return lax.rsqrt(deg)


def _psum(p):
    return p[0, :N, :] + p[1, :N, :]


def _tc_pre_body(x_ref, w_ref, degp_ref, h_ref, hs_ref):
    dinv = _dinv_from_degp(degp_ref[...])
    h = jnp.dot(x_ref[...], w_ref[...], preferred_element_type=jnp.float32)
    h_ref[...] = h
    hs_ref[...] = h * dinv


def _tc_mid_body(h_ref, p_ref, degp_ref, b_ref, g_ref, be_ref, w_ref,
                 hn_ref, hsn_ref):
    dinv = _dinv_from_degp(degp_ref[...])
    agg = dinv * _psum(p_ref[...]) + (dinv * dinv) * h_ref[...] + b_ref[...]
    mean = jnp.mean(agg, axis=0, keepdims=True)
    var = jnp.mean((agg - mean) ** 2, axis=0, keepdims=True)
    y = (agg - mean) * lax.rsqrt(var + 1e-5) * g_ref[...] + be_ref[...]
    y = jnp.maximum(y, 0.0)
    hn = jnp.dot(y, w_ref[...], preferred_element_type=jnp.float32)
    hn_ref[...] = hn
    hsn_ref[...] = hn * dinv


def _tc_fin_body(h_ref, p_ref, degp_ref, b_ref, g_ref, be_ref,
                 wl_ref, bl_ref, out_ref):
    dinv = _dinv_from_degp(degp_ref[...])
    agg = dinv * _psum(p_ref[...]) + (dinv * dinv) * h_ref[...] + b_ref[...]
    mean = jnp.mean(agg, axis=0, keepdims=True)
    var = jnp.mean((agg - mean) ** 2, axis=0, keepdims=True)
    y = (agg - mean) * lax.rsqrt(var + 1e-5) * g_ref[...] + be_ref[...]
    y = jnp.maximum(y, 0.0)
    out_ref[...] = (
        jnp.dot(y, wl_ref[...], preferred_element_type=jnp.float32)
        + bl_ref[...]
    )


_CP = pltpu.CompilerParams(vmem_limit_bytes=100 * 1024 * 1024)

_tc_pre = pl.pallas_call(
    _tc_pre_body,
    out_shape=[jax.ShapeDtypeStruct((N, D), jnp.float32),
               jax.ShapeDtypeStruct((N, D), jnp.float32)],
    compiler_params=_CP,
)

_tc_mid = pl.pallas_call(
    _tc_mid_body,
    out_shape=[jax.ShapeDtypeStruct((N, D), jnp.float32),
               jax.ShapeDtypeStruct((N, D), jnp.float32)],
    compiler_params=_CP,
)

_tc_fin = pl.pallas_call(
    _tc_fin_body,
    out_shape=jax.ShapeDtypeStruct((N, 1), jnp.float32),
    compiler_params=_CP,
)


def kernel(x, edge_index, W1, b1, g1, be1, W2, b2, g2, be2,
           W3, b3, g3, be3, Wl, bl):
    npadt = EPTP - EPT
    rows = edge_index[0].reshape(NW, EPT)
    cols = edge_index[1].reshape(NW, EPT)
    rows_p = jnp.pad(rows, ((0, 0), (0, npadt)))
    cols_p = jnp.pad(cols, ((0, 0), (0, npadt)), constant_values=TRASH)
    row3d = rows_p.reshape(NW, NCHUNK, C)
    col3d = cols_p.reshape(NW, NCHUNK, C)

    degp = _deg_kernel(col3d)

    h1, hs1 = _tc_pre(x, W1, degp)
    p1 = _agg_kernel(hs1, row3d, col3d)
    h2, hs2 = _tc_mid(h1, p1, degp, b1.reshape(1, D), g1.reshape(1, D),
                      be1.reshape(1, D), W2)
    p2 = _agg_kernel(hs2, row3d, col3d)
    h3, hs3 = _tc_mid(h2, p2, degp, b2.reshape(1, D), g2.reshape(1, D),
                      be2.reshape(1, D), W3)
    p3 = _agg_kernel(hs3, row3d, col3d)
    out = _tc_fin(h3, p3, degp, b3.reshape(1, D), g3.reshape(1, D),
                  be3.reshape(1, D), Wl, bl.reshape(1, 1))
    return out[:, 0]


# speedup vs baseline: 1.0661x; 1.0661x over previous
import functools

import jax
import jax.numpy as jnp
from jax import lax
from jax.experimental import pallas as pl
from jax.experimental.pallas import tpu as pltpu, tpu_sc as plsc

N = 10000
E = 320000
D = 128

NC = 2
NS = 16
NW = NC * NS
L = 16

EPT = E // NW
C = 128
EPTP = 10240
NCHUNK = EPTP // C
NPAIR = NCHUNK // 2 - 1
ZROWS = 8
NPAD = 10240
RPT = NPAD // NS
TRASH = N

_mesh = plsc.VectorSubcoreMesh(
    core_axis_name="c", subcore_axis_name="s", num_cores=NC, num_subcores=NS
)


@functools.partial(
    pl.kernel,
    out_type=jax.ShapeDtypeStruct((NC, NPAD, L), jnp.float32),
    mesh=_mesh,
    scratch_types=[
        pltpu.VMEM((NCHUNK, C), jnp.int32),
        pltpu.VMEM((C, L), jnp.float32),
        pltpu.VMEM((ZROWS, L), jnp.float32),
        pltpu.VMEM_SHARED((NPAD, L), jnp.float32),
    ],
)
def _deg_kernel(col_hbm, out_hbm, cidx_v, ones_v, z_v, acc):
    c = lax.axis_index("c")
    s = lax.axis_index("s")
    wid = s * NC + c
    for i in range(C):
        ones_v[i, :] = jnp.ones((L,), jnp.float32)
    for i in range(ZROWS):
        z_v[i, :] = jnp.zeros((L,), jnp.float32)
    for r in range(RPT // ZROWS):
        pltpu.sync_copy(z_v, acc.at[pl.ds(s * RPT + r * ZROWS, ZROWS)])
    pltpu.sync_copy(col_hbm.at[wid], cidx_v)
    plsc.subcore_barrier()

    def body(j, carry):
        pltpu.sync_copy(ones_v, acc.at[cidx_v.at[j]], add=True)
        return carry

    lax.fori_loop(0, NCHUNK, body, 0)
    plsc.subcore_barrier()
    pltpu.sync_copy(acc.at[pl.ds(s * RPT, RPT)],
                    out_hbm.at[c, pl.ds(s * RPT, RPT)])


@functools.partial(
    pl.kernel,
    out_type=jax.ShapeDtypeStruct((NC, NPAD, D), jnp.float32),
    mesh=_mesh,
    scratch_types=[
        pltpu.VMEM((NCHUNK, C), jnp.int32),
        pltpu.VMEM((NCHUNK, C), jnp.int32),
        pltpu.VMEM((C, D), jnp.float32),
        pltpu.VMEM((ZROWS, D), jnp.float32),
        pltpu.VMEM_SHARED((NPAD, D), jnp.float32),
        pltpu.SemaphoreType.DMA,
    ],
)
def _agg_kernel(hs_hbm, row_hbm, col_hbm, out_hbm,
                ridx_v, cidx_v, rows0, z_v, acc, sem0):
    c = lax.axis_index("c")
    s = lax.axis_index("s")
    wid = s * NC + c
    for i in range(ZROWS):
        for j in range(D // L):
            z_v[i, j * L:(j + 1) * L] = jnp.zeros((L,), jnp.float32)
    for r in range(RPT // ZROWS):
        pltpu.sync_copy(z_v, acc.at[pl.ds(s * RPT + r * ZROWS, ZROWS)])
    pltpu.sync_copy(row_hbm.at[wid], ridx_v)
    pltpu.sync_copy(col_hbm.at[wid], cidx_v)
    plsc.subcore_barrier()

    def body(j, carry):
        pltpu.async_copy(hs_hbm.at[ridx_v.at[j]], rows0, sem0).wait()
        pltpu.sync_copy(rows0, acc.at[cidx_v.at[j]], add=True)
        return carry

    lax.fori_loop(0, NCHUNK, body, 0)
    plsc.subcore_barrier()
    pltpu.sync_copy(acc.at[pl.ds(s * RPT, RPT)],
                    out_hbm.at[c, pl.ds(s * RPT, RPT)])


def _dinv_from_degp(degp):
    deg = degp[0, :N, 0:1] + degp[1, :N, 0:1] + 1.0
    return lax.rsqrt(deg)


def _psum(p):
    return p[0, :N, :] + p[1, :N, :]


def _tc_pre_body(x_ref, w_ref, degp_ref, h_ref, hs_ref):
    dinv = _dinv_from_degp(degp_ref[...])
    h = jnp.dot(x_ref[...], w_ref[...], preferred_element_type=jnp.float32)
    h_ref[...] = h
    hs_ref[...] = h * dinv


def _tc_mid_body(h_ref, p_ref, degp_ref, b_ref, g_ref, be_ref, w_ref,
                 hn_ref, hsn_ref):
    dinv = _dinv_from_degp(degp_ref[...])
    agg = dinv * _psum(p_ref[...]) + (dinv * dinv) * h_ref[...] + b_ref[...]
    mean = jnp.mean(agg, axis=0, keepdims=True)
    var = jnp.mean((agg - mean) ** 2, axis=0, keepdims=True)
    y = (agg - mean) * lax.rsqrt(var + 1e-5) * g_ref[...] + be_ref[...]
    y = jnp.maximum(y, 0.0)
    hn = jnp.dot(y, w_ref[...], preferred_element_type=jnp.float32)
    hn_ref[...] = hn
    hsn_ref[...] = hn * dinv


def _tc_fin_body(h_ref, p_ref, degp_ref, b_ref, g_ref, be_ref,
                 wl_ref, bl_ref, out_ref):
    dinv = _dinv_from_degp(degp_ref[...])
    agg = dinv * _psum(p_ref[...]) + (dinv * dinv) * h_ref[...] + b_ref[...]
    mean = jnp.mean(agg, axis=0, keepdims=True)
    var = jnp.mean((agg - mean) ** 2, axis=0, keepdims=True)
    y = (agg - mean) * lax.rsqrt(var + 1e-5) * g_ref[...] + be_ref[...]
    y = jnp.maximum(y, 0.0)
    out_ref[...] = (
        jnp.dot(y, wl_ref[...], preferred_element_type=jnp.float32)
        + bl_ref[...]
    )


_CP = pltpu.CompilerParams(vmem_limit_bytes=100 * 1024 * 1024)

_tc_pre = pl.pallas_call(
    _tc_pre_body,
    out_shape=[jax.ShapeDtypeStruct((N, D), jnp.float32),
               jax.ShapeDtypeStruct((N, D), jnp.float32)],
    compiler_params=_CP,
)

_tc_mid = pl.pallas_call(
    _tc_mid_body,
    out_shape=[jax.ShapeDtypeStruct((N, D), jnp.float32),
               jax.ShapeDtypeStruct((N, D), jnp.float32)],
    compiler_params=_CP,
)

_tc_fin = pl.pallas_call(
    _tc_fin_body,
    out_shape=jax.ShapeDtypeStruct((N, 1), jnp.float32),
    compiler_params=_CP,
)


def kernel(x, edge_index, W1, b1, g1, be1, W2, b2, g2, be2,
           W3, b3, g3, be3, Wl, bl):
    npadt = EPTP - EPT
    rows = edge_index[0].reshape(NW, EPT)
    cols = edge_index[1].reshape(NW, EPT)
    rows_p = jnp.pad(rows, ((0, 0), (0, npadt)))
    trash = jnp.broadcast_to(TRASH + jnp.arange(npadt, dtype=cols.dtype),
                             (NW, npadt))
    cols_p = jnp.concatenate([cols, trash], axis=1)
    row3d = rows_p.reshape(NW, NCHUNK, C)
    col3d = cols_p.reshape(NW, NCHUNK, C)

    degp = _deg_kernel(col3d)

    h1, hs1 = _tc_pre(x, W1, degp)
    p1 = _agg_kernel(hs1, row3d, col3d)
    h2, hs2 = _tc_mid(h1, p1, degp, b1.reshape(1, D), g1.reshape(1, D),
                      be1.reshape(1, D), W2)
    p2 = _agg_kernel(hs2, row3d, col3d)
    h3, hs3 = _tc_mid(h2, p2, degp, b2.reshape(1, D), g2.reshape(1, D),
                      be2.reshape(1, D), W3)
    p3 = _agg_kernel(hs3, row3d, col3d)
    out = _tc_fin(h3, p3, degp, b3.reshape(1, D), g3.reshape(1, D),
                  be3.reshape(1, D), Wl, bl.reshape(1, 1))
    return out[:, 0]


# speedup vs baseline: 1.0756x; 1.0089x over previous
import functools

import jax
import jax.numpy as jnp
from jax import lax
from jax.experimental import pallas as pl
from jax.experimental.pallas import tpu as pltpu, tpu_sc as plsc

N = 10000
E = 320000
D = 128

NC = 2
NS = 16
NW = NC * NS
L = 16

EPT = E // NW
C = 128
EPTP = 10240
NCHUNK = EPTP // C
NPAIR = NCHUNK // 2 - 1
ZROWS = 8
NPAD = 10240
RPT = NPAD // NS
TRASH = N

_mesh = plsc.VectorSubcoreMesh(
    core_axis_name="c", subcore_axis_name="s", num_cores=NC, num_subcores=NS
)


@functools.partial(
    pl.kernel,
    out_type=jax.ShapeDtypeStruct((NC, NPAD, L), jnp.float32),
    mesh=_mesh,
    scratch_types=[
        pltpu.VMEM((NCHUNK, C), jnp.int32),
        pltpu.VMEM((C, L), jnp.float32),
        pltpu.VMEM((ZROWS, L), jnp.float32),
        pltpu.VMEM_SHARED((NPAD, L), jnp.float32),
    ],
)
def _deg_kernel(col_hbm, out_hbm, cidx_v, ones_v, z_v, acc):
    c = lax.axis_index("c")
    s = lax.axis_index("s")
    wid = s * NC + c
    for i in range(C):
        ones_v[i, :] = jnp.ones((L,), jnp.float32)
    for i in range(ZROWS):
        z_v[i, :] = jnp.zeros((L,), jnp.float32)
    for r in range(RPT // ZROWS):
        pltpu.sync_copy(z_v, acc.at[pl.ds(s * RPT + r * ZROWS, ZROWS)])
    pltpu.sync_copy(col_hbm.at[wid], cidx_v)
    plsc.subcore_barrier()

    def body(j, carry):
        pltpu.sync_copy(ones_v, acc.at[cidx_v.at[j]], add=True)
        return carry

    lax.fori_loop(0, NCHUNK, body, 0)
    plsc.subcore_barrier()
    pltpu.sync_copy(acc.at[pl.ds(s * RPT, RPT)],
                    out_hbm.at[c, pl.ds(s * RPT, RPT)])


@functools.partial(
    pl.kernel,
    out_type=jax.ShapeDtypeStruct((NC, NPAD, D), jnp.float32),
    mesh=_mesh,
    scratch_types=[
        pltpu.VMEM((NCHUNK, C), jnp.int32),
        pltpu.VMEM((NCHUNK, C), jnp.int32),
        pltpu.VMEM((C, D), jnp.float32),
        pltpu.VMEM_SHARED((NPAD, D), jnp.float32),
        pltpu.SemaphoreType.DMA,
    ],
)
def _agg_kernel(hs_hbm, row_hbm, col_hbm, out_hbm,
                ridx_v, cidx_v, rows0, acc, sem0):
    c = lax.axis_index("c")
    s = lax.axis_index("s")
    wid = s * NC + c

    def zbody(i, carry):
        for j in range(D // L):
            rows0[i, j * L:(j + 1) * L] = jnp.zeros((L,), jnp.float32)
        return carry

    lax.fori_loop(0, C, zbody, 0)
    for r in range(RPT // C):
        pltpu.sync_copy(rows0, acc.at[pl.ds(s * RPT + r * C, C)])
    pltpu.sync_copy(row_hbm.at[wid], ridx_v)
    pltpu.sync_copy(col_hbm.at[wid], cidx_v)
    plsc.subcore_barrier()

    def body(j, carry):
        pltpu.async_copy(hs_hbm.at[ridx_v.at[j]], rows0, sem0).wait()
        pltpu.sync_copy(rows0, acc.at[cidx_v.at[j]], add=True)
        return carry

    lax.fori_loop(0, NCHUNK, body, 0)
    plsc.subcore_barrier()
    pltpu.sync_copy(acc.at[pl.ds(s * RPT, RPT)],
                    out_hbm.at[c, pl.ds(s * RPT, RPT)])


def _dinv_from_degp(degp):
    deg = degp[0, :N, 0:1] + degp[1, :N, 0:1] + 1.0
    return lax.rsqrt(deg)


def _psum(p):
    return p[0, :N, :] + p[1, :N, :]


def _tc_pre_body(x_ref, w_ref, degp_ref, h_ref, hs_ref):
    dinv = _dinv_from_degp(degp_ref[...])
    h = jnp.dot(x_ref[...], w_ref[...], preferred_element_type=jnp.float32)
    h_ref[...] = h
    hs_ref[...] = h * dinv


def _tc_mid_body(h_ref, p_ref, degp_ref, b_ref, g_ref, be_ref, w_ref,
                 hn_ref, hsn_ref):
    dinv = _dinv_from_degp(degp_ref[...])
    agg = dinv * _psum(p_ref[...]) + (dinv * dinv) * h_ref[...] + b_ref[...]
    mean = jnp.mean(agg, axis=0, keepdims=True)
    var = jnp.mean((agg - mean) ** 2, axis=0, keepdims=True)
    y = (agg - mean) * lax.rsqrt(var + 1e-5) * g_ref[...] + be_ref[...]
    y = jnp.maximum(y, 0.0)
    hn = jnp.dot(y, w_ref[...], preferred_element_type=jnp.float32)
    hn_ref[...] = hn
    hsn_ref[...] = hn * dinv


def _tc_fin_body(h_ref, p_ref, degp_ref, b_ref, g_ref, be_ref,
                 wl_ref, bl_ref, out_ref):
    dinv = _dinv_from_degp(degp_ref[...])
    agg = dinv * _psum(p_ref[...]) + (dinv * dinv) * h_ref[...] + b_ref[...]
    mean = jnp.mean(agg, axis=0, keepdims=True)
    var = jnp.mean((agg - mean) ** 2, axis=0, keepdims=True)
    y = (agg - mean) * lax.rsqrt(var + 1e-5) * g_ref[...] + be_ref[...]
    y = jnp.maximum(y, 0.0)
    out_ref[...] = (
        jnp.dot(y, wl_ref[...], preferred_element_type=jnp.float32)
        + bl_ref[...]
    )


_CP = pltpu.CompilerParams(vmem_limit_bytes=100 * 1024 * 1024)

_tc_pre = pl.pallas_call(
    _tc_pre_body,
    out_shape=[jax.ShapeDtypeStruct((N, D), jnp.float32),
               jax.ShapeDtypeStruct((N, D), jnp.float32)],
    compiler_params=_CP,
)

_tc_mid = pl.pallas_call(
    _tc_mid_body,
    out_shape=[jax.ShapeDtypeStruct((N, D), jnp.float32),
               jax.ShapeDtypeStruct((N, D), jnp.float32)],
    compiler_params=_CP,
)

_tc_fin = pl.pallas_call(
    _tc_fin_body,
    out_shape=jax.ShapeDtypeStruct((N, 1), jnp.float32),
    compiler_params=_CP,
)


def kernel(x, edge_index, W1, b1, g1, be1, W2, b2, g2, be2,
           W3, b3, g3, be3, Wl, bl):
    npadt = EPTP - EPT
    rows = edge_index[0].reshape(NW, EPT)
    cols = edge_index[1].reshape(NW, EPT)
    rows_p = jnp.pad(rows, ((0, 0), (0, npadt)))
    trash = jnp.broadcast_to(TRASH + jnp.arange(npadt, dtype=cols.dtype),
                             (NW, npadt))
    cols_p = jnp.concatenate([cols, trash], axis=1)
    row3d = rows_p.reshape(NW, NCHUNK, C)
    col3d = cols_p.reshape(NW, NCHUNK, C)

    degp = _deg_kernel(col3d)

    h1, hs1 = _tc_pre(x, W1, degp)
    p1 = _agg_kernel(hs1, row3d, col3d)
    h2, hs2 = _tc_mid(h1, p1, degp, b1.reshape(1, D), g1.reshape(1, D),
                      be1.reshape(1, D), W2)
    p2 = _agg_kernel(hs2, row3d, col3d)
    h3, hs3 = _tc_mid(h2, p2, degp, b2.reshape(1, D), g2.reshape(1, D),
                      be2.reshape(1, D), W3)
    p3 = _agg_kernel(hs3, row3d, col3d)
    out = _tc_fin(h3, p3, degp, b3.reshape(1, D), g3.reshape(1, D),
                  be3.reshape(1, D), Wl, bl.reshape(1, 1))
    return out[:, 0]


# speedup vs baseline: 3.3724x; 3.1353x over previous
import functools

import jax
import jax.numpy as jnp
from jax import lax
from jax.experimental import pallas as pl
from jax.experimental.pallas import tpu as pltpu, tpu_sc as plsc

N = 10000
E = 320000
D = 128

NC = 2
NS = 16
NW = NC * NS
L = 16

EPT = E // NW
C = 80
NCHUNK = EPT // C
NPAIR = (NCHUNK - 1) // 2
NPAD = 10240
RPT = NPAD // NS
ZROWS = 32

_mesh = plsc.VectorSubcoreMesh(
    core_axis_name="c", subcore_axis_name="s", num_cores=NC, num_subcores=NS
)


@functools.partial(
    pl.kernel,
    out_type=jax.ShapeDtypeStruct((NC, NPAD, L), jnp.float32),
    mesh=_mesh,
    scratch_types=[
        pltpu.VMEM((NCHUNK, C), jnp.int32),
        pltpu.VMEM((C, L), jnp.float32),
        pltpu.VMEM((ZROWS, L), jnp.float32),
        pltpu.VMEM_SHARED((NPAD, L), jnp.float32),
    ],
)
def _deg_kernel(col_hbm, out_hbm, cidx_v, ones_v, z_v, acc):
    c = lax.axis_index("c")
    s = lax.axis_index("s")
    wid = s * NC + c
    for i in range(C):
        ones_v[i, :] = jnp.ones((L,), jnp.float32)
    for i in range(ZROWS):
        z_v[i, :] = jnp.zeros((L,), jnp.float32)
    for r in range(RPT // ZROWS):
        pltpu.sync_copy(z_v, acc.at[pl.ds(s * RPT + r * ZROWS, ZROWS)])
    pltpu.sync_copy(col_hbm.at[wid], cidx_v)
    plsc.subcore_barrier()

    def body(j, carry):
        pltpu.sync_copy(ones_v, acc.at[cidx_v.at[j]], add=True)
        return carry

    lax.fori_loop(0, NCHUNK, body, 0)
    plsc.subcore_barrier()
    pltpu.sync_copy(acc.at[pl.ds(s * RPT, RPT)],
                    out_hbm.at[c, pl.ds(s * RPT, RPT)])


@functools.partial(
    pl.kernel,
    out_type=jax.ShapeDtypeStruct((NC, NPAD, D), jnp.float32),
    mesh=_mesh,
    scratch_types=[
        pltpu.VMEM((EPT,), jnp.int32),
        pltpu.VMEM((NCHUNK, C), jnp.int32),
        pltpu.VMEM((C, D), jnp.float32),
        pltpu.VMEM((C, D), jnp.float32),
        pltpu.VMEM_SHARED((NPAD, D), jnp.float32),
        pltpu.SemaphoreType.DMA,
        pltpu.SemaphoreType.DMA,
    ],
)
def _agg_kernel(hs_hbm, row_hbm, col_hbm, out_hbm,
                ridx_v, cidx_v, rows0, rows1, acc, sem0, sem1):
    c = lax.axis_index("c")
    s = lax.axis_index("s")
    wid = s * NC + c

    def zbody(i, carry):
        for j in range(D // L):
            rows0[i, j * L:(j + 1) * L] = jnp.zeros((L,), jnp.float32)
        return carry

    lax.fori_loop(0, C, zbody, 0)
    for r in range(RPT // C):
        pltpu.sync_copy(rows0, acc.at[pl.ds(s * RPT + r * C, C)])
    pltpu.sync_copy(row_hbm.at[wid], ridx_v)
    pltpu.sync_copy(col_hbm.at[wid], cidx_v)
    plsc.subcore_barrier()

    def g(j, buf, sem):
        pltpu.async_copy(hs_hbm.at[ridx_v.at[pl.ds(j * C, C)]], buf, sem)

    def w(j, buf, sem):
        pltpu.make_async_copy(hs_hbm.at[ridx_v.at[pl.ds(j * C, C)]],
                              buf, sem).wait()

    g(0, rows0, sem0)

    def body(k, carry):
        j = 2 * k
        g(j + 1, rows1, sem1)
        w(j, rows0, sem0)
        pltpu.sync_copy(rows0, acc.at[cidx_v.at[j]], add=True)
        g(j + 2, rows0, sem0)
        w(j + 1, rows1, sem1)
        pltpu.sync_copy(rows1, acc.at[cidx_v.at[j + 1]], add=True)
        return carry

    lax.fori_loop(0, NPAIR, body, 0)
    j = NCHUNK - 1
    w(j, rows0, sem0)
    pltpu.sync_copy(rows0, acc.at[cidx_v.at[j]], add=True)
    plsc.subcore_barrier()
    pltpu.sync_copy(acc.at[pl.ds(s * RPT, RPT)],
                    out_hbm.at[c, pl.ds(s * RPT, RPT)])


def _dinv_from_degp(degp):
    deg = degp[0, :N, 0:1] + degp[1, :N, 0:1] + 1.0
    return lax.rsqrt(deg)


def _psum(p):
    return p[0, :N, :] + p[1, :N, :]


def _tc_pre_body(x_ref, w_ref, degp_ref, h_ref, hs_ref):
    dinv = _dinv_from_degp(degp_ref[...])
    h = jnp.dot(x_ref[...], w_ref[...], preferred_element_type=jnp.float32)
    h_ref[...] = h
    hs_ref[...] = h * dinv


def _tc_mid_body(h_ref, p_ref, degp_ref, b_ref, g_ref, be_ref, w_ref,
                 hn_ref, hsn_ref):
    dinv = _dinv_from_degp(degp_ref[...])
    agg = dinv * _psum(p_ref[...]) + (dinv * dinv) * h_ref[...] + b_ref[...]
    mean = jnp.mean(agg, axis=0, keepdims=True)
    var = jnp.mean((agg - mean) ** 2, axis=0, keepdims=True)
    y = (agg - mean) * lax.rsqrt(var + 1e-5) * g_ref[...] + be_ref[...]
    y = jnp.maximum(y, 0.0)
    hn = jnp.dot(y, w_ref[...], preferred_element_type=jnp.float32)
    hn_ref[...] = hn
    hsn_ref[...] = hn * dinv


def _tc_fin_body(h_ref, p_ref, degp_ref, b_ref, g_ref, be_ref,
                 wl_ref, bl_ref, out_ref):
    dinv = _dinv_from_degp(degp_ref[...])
    agg = dinv * _psum(p_ref[...]) + (dinv * dinv) * h_ref[...] + b_ref[...]
    mean = jnp.mean(agg, axis=0, keepdims=True)
    var = jnp.mean((agg - mean) ** 2, axis=0, keepdims=True)
    y = (agg - mean) * lax.rsqrt(var + 1e-5) * g_ref[...] + be_ref[...]
    y = jnp.maximum(y, 0.0)
    out_ref[...] = (
        jnp.dot(y, wl_ref[...], preferred_element_type=jnp.float32)
        + bl_ref[...]
    )


_CP = pltpu.CompilerParams(vmem_limit_bytes=100 * 1024 * 1024)

_tc_pre = pl.pallas_call(
    _tc_pre_body,
    out_shape=[jax.ShapeDtypeStruct((N, D), jnp.float32),
               jax.ShapeDtypeStruct((N, D), jnp.float32)],
    compiler_params=_CP,
)

_tc_mid = pl.pallas_call(
    _tc_mid_body,
    out_shape=[jax.ShapeDtypeStruct((N, D), jnp.float32),
               jax.ShapeDtypeStruct((N, D), jnp.float32)],
    compiler_params=_CP,
)

_tc_fin = pl.pallas_call(
    _tc_fin_body,
    out_shape=jax.ShapeDtypeStruct((N, 1), jnp.float32),
    compiler_params=_CP,
)


def kernel(x, edge_index, W1, b1, g1, be1, W2, b2, g2, be2,
           W3, b3, g3, be3, Wl, bl):
    row2d = edge_index[0].reshape(NW, EPT)
    col3d = edge_index[1].reshape(NW, NCHUNK, C)

    degp = _deg_kernel(col3d)

    h1, hs1 = _tc_pre(x, W1, degp)
    p1 = _agg_kernel(hs1, row2d, col3d)
    h2, hs2 = _tc_mid(h1, p1, degp, b1.reshape(1, D), g1.reshape(1, D),
                      be1.reshape(1, D), W2)
    p2 = _agg_kernel(hs2, row2d, col3d)
    h3, hs3 = _tc_mid(h2, p2, degp, b2.reshape(1, D), g2.reshape(1, D),
                      be2.reshape(1, D), W3)
    p3 = _agg_kernel(hs3, row2d, col3d)
    out = _tc_fin(h3, p3, degp, b3.reshape(1, D), g3.reshape(1, D),
                  be3.reshape(1, D), Wl, bl.reshape(1, 1))
    return out[:, 0]


# speedup vs baseline: 3.4194x; 1.0139x over previous
import functools

import jax
import jax.numpy as jnp
from jax import lax
from jax.experimental import pallas as pl
from jax.experimental.pallas import tpu as pltpu, tpu_sc as plsc

N = 10000
E = 320000
D = 128

NC = 2
NS = 16
NW = NC * NS
L = 16

EPT = E // NW
C = 80
NCHUNK = EPT // C
NPAIR = (NCHUNK - 1) // 2
NPAD = 10240
RPT = NPAD // NS
ZROWS = 32

_mesh = plsc.VectorSubcoreMesh(
    core_axis_name="c", subcore_axis_name="s", num_cores=NC, num_subcores=NS
)


@functools.partial(
    pl.kernel,
    out_type=jax.ShapeDtypeStruct((NC, NPAD, L), jnp.float32),
    mesh=_mesh,
    scratch_types=[
        pltpu.VMEM((NCHUNK, C), jnp.int32),
        pltpu.VMEM((C, L), jnp.float32),
        pltpu.VMEM((ZROWS, L), jnp.float32),
        pltpu.VMEM_SHARED((NPAD, L), jnp.float32),
        pltpu.SemaphoreType.DMA,
    ],
)
def _deg_kernel(col_hbm, out_hbm, cidx_v, ones_v, z_v, acc, sem):
    c = lax.axis_index("c")
    s = lax.axis_index("s")
    wid = s * NC + c
    for i in range(C):
        ones_v[i, :] = jnp.ones((L,), jnp.float32)
    for i in range(ZROWS):
        z_v[i, :] = jnp.zeros((L,), jnp.float32)
    for r in range(RPT // ZROWS):
        pltpu.sync_copy(z_v, acc.at[pl.ds(s * RPT + r * ZROWS, ZROWS)])
    pltpu.sync_copy(col_hbm.at[wid], cidx_v)
    plsc.subcore_barrier()

    def body(gidx, carry):
        j = gidx * 5
        for t in range(5):
            pltpu.async_copy(ones_v, acc.at[cidx_v.at[j + t]], sem, add=True)
        for t in range(5):
            pltpu.make_async_copy(ones_v, acc.at[cidx_v.at[j + t]],
                                  sem).wait()
        return carry

    lax.fori_loop(0, NCHUNK // 5, body, 0)
    plsc.subcore_barrier()
    pltpu.sync_copy(acc.at[pl.ds(s * RPT, RPT)],
                    out_hbm.at[c, pl.ds(s * RPT, RPT)])


@functools.partial(
    pl.kernel,
    out_type=jax.ShapeDtypeStruct((NC, NPAD, D), jnp.float32),
    mesh=_mesh,
    scratch_types=[
        pltpu.VMEM((EPT,), jnp.int32),
        pltpu.VMEM((NCHUNK, C), jnp.int32),
        pltpu.VMEM((C, D), jnp.float32),
        pltpu.VMEM((C, D), jnp.float32),
        pltpu.VMEM_SHARED((NPAD, D), jnp.float32),
        pltpu.SemaphoreType.DMA,
        pltpu.SemaphoreType.DMA,
    ],
)
def _agg_kernel(hs_hbm, row_hbm, col_hbm, out_hbm,
                ridx_v, cidx_v, rows0, rows1, acc, sem0, sem1):
    c = lax.axis_index("c")
    s = lax.axis_index("s")
    wid = s * NC + c

    def zbody(i, carry):
        for j in range(D // L):
            rows0[i, j * L:(j + 1) * L] = jnp.zeros((L,), jnp.float32)
        return carry

    lax.fori_loop(0, C, zbody, 0)
    for r in range(RPT // C):
        pltpu.sync_copy(rows0, acc.at[pl.ds(s * RPT + r * C, C)])
    pltpu.sync_copy(row_hbm.at[wid], ridx_v)
    pltpu.sync_copy(col_hbm.at[wid], cidx_v)
    plsc.subcore_barrier()

    def g(j, buf, sem):
        pltpu.async_copy(hs_hbm.at[ridx_v.at[pl.ds(j * C, C)]], buf, sem)

    def w(j, buf, sem):
        pltpu.make_async_copy(hs_hbm.at[ridx_v.at[pl.ds(j * C, C)]],
                              buf, sem).wait()

    g(0, rows0, sem0)

    def body(k, carry):
        j = 2 * k
        g(j + 1, rows1, sem1)
        w(j, rows0, sem0)
        pltpu.sync_copy(rows0, acc.at[cidx_v.at[j]], add=True)
        g(j + 2, rows0, sem0)
        w(j + 1, rows1, sem1)
        pltpu.sync_copy(rows1, acc.at[cidx_v.at[j + 1]], add=True)
        return carry

    lax.fori_loop(0, NPAIR, body, 0)
    j = NCHUNK - 1
    w(j, rows0, sem0)
    pltpu.sync_copy(rows0, acc.at[cidx_v.at[j]], add=True)
    plsc.subcore_barrier()
    pltpu.sync_copy(acc.at[pl.ds(s * RPT, RPT)],
                    out_hbm.at[c, pl.ds(s * RPT, RPT)])


def _dinv_from_degp(degp):
    deg = degp[0, :N, 0:1] + degp[1, :N, 0:1] + 1.0
    return lax.rsqrt(deg)


def _psum(p):
    return p[0, :N, :] + p[1, :N, :]


def _tc_pre_body(x_ref, w_ref, degp_ref, h_ref, hs_ref):
    dinv = _dinv_from_degp(degp_ref[...])
    h = jnp.dot(x_ref[...], w_ref[...], preferred_element_type=jnp.float32)
    h_ref[...] = h
    hs_ref[...] = h * dinv


def _tc_mid_body(h_ref, p_ref, degp_ref, b_ref, g_ref, be_ref, w_ref,
                 hn_ref, hsn_ref):
    dinv = _dinv_from_degp(degp_ref[...])
    agg = dinv * _psum(p_ref[...]) + (dinv * dinv) * h_ref[...] + b_ref[...]
    mean = jnp.mean(agg, axis=0, keepdims=True)
    var = jnp.mean((agg - mean) ** 2, axis=0, keepdims=True)
    y = (agg - mean) * lax.rsqrt(var + 1e-5) * g_ref[...] + be_ref[...]
    y = jnp.maximum(y, 0.0)
    hn = jnp.dot(y, w_ref[...], preferred_element_type=jnp.float32)
    hn_ref[...] = hn
    hsn_ref[...] = hn * dinv


def _tc_fin_body(h_ref, p_ref, degp_ref, b_ref, g_ref, be_ref,
                 wl_ref, bl_ref, out_ref):
    dinv = _dinv_from_degp(degp_ref[...])
    agg = dinv * _psum(p_ref[...]) + (dinv * dinv) * h_ref[...] + b_ref[...]
    mean = jnp.mean(agg, axis=0, keepdims=True)
    var = jnp.mean((agg - mean) ** 2, axis=0, keepdims=True)
    y = (agg - mean) * lax.rsqrt(var + 1e-5) * g_ref[...] + be_ref[...]
    y = jnp.maximum(y, 0.0)
    out_ref[...] = (
        jnp.dot(y, wl_ref[...], preferred_element_type=jnp.float32)
        + bl_ref[...]
    )


_CP = pltpu.CompilerParams(vmem_limit_bytes=100 * 1024 * 1024)

_tc_pre = pl.pallas_call(
    _tc_pre_body,
    out_shape=[jax.ShapeDtypeStruct((N, D), jnp.float32),
               jax.ShapeDtypeStruct((N, D), jnp.float32)],
    compiler_params=_CP,
)

_tc_mid = pl.pallas_call(
    _tc_mid_body,
    out_shape=[jax.ShapeDtypeStruct((N, D), jnp.float32),
               jax.ShapeDtypeStruct((N, D), jnp.float32)],
    compiler_params=_CP,
)

_tc_fin = pl.pallas_call(
    _tc_fin_body,
    out_shape=jax.ShapeDtypeStruct((N, 1), jnp.float32),
    compiler_params=_CP,
)


def kernel(x, edge_index, W1, b1, g1, be1, W2, b2, g2, be2,
           W3, b3, g3, be3, Wl, bl):
    row2d = edge_index[0].reshape(NW, EPT)
    col3d = edge_index[1].reshape(NW, NCHUNK, C)

    degp = _deg_kernel(col3d)

    h1, hs1 = _tc_pre(x, W1, degp)
    p1 = _agg_kernel(hs1, row2d, col3d)
    h2, hs2 = _tc_mid(h1, p1, degp, b1.reshape(1, D), g1.reshape(1, D),
                      be1.reshape(1, D), W2)
    p2 = _agg_kernel(hs2, row2d, col3d)
    h3, hs3 = _tc_mid(h2, p2, degp, b2.reshape(1, D), g2.reshape(1, D),
                      be2.reshape(1, D), W3)
    p3 = _agg_kernel(hs3, row2d, col3d)
    out = _tc_fin(h3, p3, degp, b3.reshape(1, D), g3.reshape(1, D),
                  be3.reshape(1, D), Wl, bl.reshape(1, 1))
    return out[:, 0]
